# baseline (speedup 1.0000x reference)
import functools

import jax
import jax.numpy as jnp
from jax import lax
from jax.experimental import pallas as pl
from jax.experimental.pallas import tpu as pltpu
from jax.experimental.pallas import tpu_sc as plsc

N = 10000
E = 320000
D = 128
OUT = 128

NC = 2
NS = 16
NW = NC * NS
EPT = E // NW
K = 80
SB = 25
NSUPER = EPT // (K * SB)
NP = 10240
RPT = NP // NS


def _t1_body(x_ref, w_ref, b_ref, a_ref, wh_ref, s_ref, c_ref):
    wh = jnp.dot(x_ref[:], w_ref[:], preferred_element_type=jnp.float32)
    wh = wh + b_ref[:]
    wh_ref[:] = wh
    s = lax.dot_general(a_ref[:], wh, (((1,), (1,)), ((), ())),
                        preferred_element_type=jnp.float32)
    s_ref[:] = s
    t = jnp.max(s[0]) + jnp.max(s[1])
    c = jnp.where(t >= 0, t, t * 0.01)
    c_ref[:] = jnp.full((1, 128), c, dtype=jnp.float32)


def _t3_body(g_ref, den_ref, h_ref):
    num = g_ref[0] + g_ref[1]
    den = jnp.sum(den_ref[0], axis=0)
    den = den[:, None]
    h_ref[:] = jnp.where(den > 0, num / den, 0.0)


def _sc_body(src4, dst4, s1_hbm, s2_hbm, c_hbm, wh_hbm, zeros_hbm,
             g_out, den_out,
             src2d_v, dst2d_v, s1_v, s2_v, c_v,
             rows_v, den_v, gsem, g_s):
    cid = lax.axis_index("c")
    sid = lax.axis_index("s")
    wid = cid * NS + sid

    pltpu.sync_copy(s1_hbm, s1_v)
    pltpu.sync_copy(s2_hbm, s2_v)
    pltpu.sync_copy(c_hbm, c_v)
    cvec = c_v[pl.ds(0, 16)]

    pltpu.sync_copy(zeros_hbm, g_s.at[pl.ds(sid * RPT, RPT)])

    def zero_body(i, carry):
        den_v[pl.ds(i * 16, 16)] = jnp.zeros((16,), jnp.float32)
        return carry

    lax.fori_loop(0, N // 16, zero_body, 0)
    plsc.subcore_barrier()

    lanes = lax.iota(jnp.int32, 16)

    def super_body(si, carry):
        pltpu.sync_copy(src4.at[wid, si], src2d_v)
        pltpu.sync_copy(dst4.at[wid, si], dst2d_v)

        def chunk_body(cj, carry2):
            gdesc = pltpu.async_copy(wh_hbm.at[src2d_v.at[cj]],
                                     rows_v, gsem)
            exs = []
            for g in range(K // 16):
                src16 = src2d_v[cj, pl.ds(g * 16, 16)]
                dst16 = dst2d_v[cj, pl.ds(g * 16, 16)]
                v1 = plsc.load_gather(s1_v, [src16])
                v2 = plsc.load_gather(s2_v, [dst16])
                t = v1 + v2
                e = jnp.where(t >= 0, t, t * 0.01)
                ex = jnp.exp(e - cvec)
                for j in range(16):
                    plsc.addupdate_scatter(den_v, [dst16], ex,
                                           mask=lanes == j)
                exs.append(ex)
            gdesc.wait()
            for g in range(K // 16):
                ex = exs[g]
                bjs = [lax.gather(
                    ex, jnp.full((16, 1), j, jnp.int32),
                    lax.GatherDimensionNumbers(
                        offset_dims=(), collapsed_slice_dims=(0,),
                        start_index_map=(0,)),
                    (1,),
                    mode=lax.GatherScatterMode.PROMISE_IN_BOUNDS)
                    for j in range(16)]
                for j in range(0, 16, 2):
                    r0 = g * 16 + j
                    r1 = r0 + 1
                    l0 = [rows_v[r0, pl.ds(q * 16, 16)]
                          for q in range(D // 16)]
                    l1 = [rows_v[r1, pl.ds(q * 16, 16)]
                          for q in range(D // 16)]
                    for q in range(D // 16):
                        rows_v[r0, pl.ds(q * 16, 16)] = l0[q] * bjs[j]
                    for q in range(D // 16):
                        rows_v[r1, pl.ds(q * 16, 16)] = l1[q] * bjs[j + 1]
            pltpu.sync_copy(rows_v, g_s.at[dst2d_v.at[cj]], add=True)
            return carry2

        lax.fori_loop(0, SB, chunk_body, 0)
        return carry

    lax.fori_loop(0, NSUPER, super_body, 0)
    plsc.subcore_barrier()

    row0 = sid * RPT
    pltpu.sync_copy(g_s.at[pl.ds(row0, RPT)],
                    g_out.at[cid, pl.ds(row0, RPT)])
    pltpu.sync_copy(den_v, den_out.at[cid, sid])


_sc_mesh = plsc.VectorSubcoreMesh(core_axis_name="c", subcore_axis_name="s",
                                  num_cores=NC, num_subcores=NS)

_sc_kernel = functools.partial(
    pl.kernel,
    out_type=(jax.ShapeDtypeStruct((NC, NP, D), jnp.float32),
              jax.ShapeDtypeStruct((NC, NS, N), jnp.float32)),
    mesh=_sc_mesh,
    scratch_types=[
        pltpu.VMEM((SB, K), jnp.int32),
        pltpu.VMEM((SB, K), jnp.int32),
        pltpu.VMEM((N,), jnp.float32),
        pltpu.VMEM((N,), jnp.float32),
        pltpu.VMEM((128,), jnp.float32),
        pltpu.VMEM((K, D), jnp.float32),
        pltpu.VMEM((N,), jnp.float32),
        pltpu.SemaphoreType.DMA,
        pltpu.VMEM_SHARED((NP, D), jnp.float32),
    ],
    compiler_params=pltpu.CompilerParams(needs_layout_passes=False),
)(_sc_body)


@jax.jit
def kernel(x, edge_index, W, b, a_w):
    src4 = edge_index[0].reshape(NW, NSUPER, SB, K)
    dst4 = edge_index[1].reshape(NW, NSUPER, SB, K)
    a2 = a_w.reshape(2, D)
    b2 = b.reshape(1, OUT)

    wh, s, c = pl.pallas_call(
        _t1_body,
        out_shape=[
            jax.ShapeDtypeStruct((N, OUT), jnp.float32),
            jax.ShapeDtypeStruct((2, N), jnp.float32),
            jax.ShapeDtypeStruct((1, 128), jnp.float32),
        ],
    )(x, W, b2, a2)

    s1 = s[0]
    s2 = s[1]
    c128 = c.reshape(128)
    zeros = jnp.zeros((RPT, D), dtype=jnp.float32)

    g, den = _sc_kernel(src4, dst4, s1, s2, c128, wh, zeros)

    BR = 1000
    h = pl.pallas_call(
        _t3_body,
        grid=(N // BR,),
        in_specs=[
            pl.BlockSpec((NC, BR, D), lambda i: (0, i, 0)),
            pl.BlockSpec((1, NC * NS, BR), lambda i: (i, 0, 0)),
        ],
        out_specs=pl.BlockSpec((BR, OUT), lambda i: (i, 0)),
        out_shape=jax.ShapeDtypeStruct((N, OUT), jnp.float32),
    )(g, den.reshape(NC * NS, N // BR, BR).transpose(1, 0, 2))
    return h


# speedup vs baseline: 1.1080x; 1.1080x over previous
import functools

import jax
import jax.numpy as jnp
from jax import lax
from jax.experimental import pallas as pl
from jax.experimental.pallas import tpu as pltpu
from jax.experimental.pallas import tpu_sc as plsc

N = 10000
E = 320000
D = 128
OUT = 128

NC = 2
NS = 16
NW = NC * NS
EPT = E // NW
K = 80
SB = 25
NSUPER = EPT // (K * SB)
NP = 10240
RPT = NP // NS


def _t1_body(x_ref, w_ref, b_ref, a_ref, wh_ref, s_ref, c_ref):
    wh = jnp.dot(x_ref[:], w_ref[:], preferred_element_type=jnp.float32)
    wh = wh + b_ref[:]
    wh_ref[:] = wh
    s = lax.dot_general(a_ref[:], wh, (((1,), (1,)), ((), ())),
                        preferred_element_type=jnp.float32)
    s_ref[:] = s
    t = jnp.max(s[0]) + jnp.max(s[1])
    c = jnp.where(t >= 0, t, t * 0.01)
    c_ref[:] = jnp.full((1, 128), c, dtype=jnp.float32)


def _t3_body(g_ref, den_ref, h_ref):
    num = g_ref[0] + g_ref[1]
    den = jnp.sum(den_ref[0], axis=0)
    den = den[:, None]
    h_ref[:] = jnp.where(den > 0, num / den, 0.0)


def _sc_body(src4, dst4, s1_hbm, s2_hbm, c_hbm, wh_hbm, zeros_hbm,
             g_out, den_out,
             src2d_v, dst2d_v, s1_v, s2_v, c_v,
             rows_v, den_v, gsem, ssem, g_s):
    cid = lax.axis_index("c")
    sid = lax.axis_index("s")
    wid = cid * NS + sid

    pltpu.sync_copy(s1_hbm, s1_v)
    pltpu.sync_copy(s2_hbm, s2_v)
    pltpu.sync_copy(c_hbm, c_v)
    cvec = c_v[pl.ds(0, 16)]

    pltpu.sync_copy(zeros_hbm, g_s.at[pl.ds(sid * RPT, RPT)])

    def zero_body(i, carry):
        den_v[pl.ds(i * 16, 16)] = jnp.zeros((16,), jnp.float32)
        return carry

    lax.fori_loop(0, N // 16, zero_body, 0)
    plsc.subcore_barrier()

    lanes = lax.iota(jnp.int32, 16)
    zeros16 = jnp.zeros((16,), jnp.int32)

    def super_body(si, carry):
        pltpu.sync_copy(src4.at[wid, si], src2d_v)
        pltpu.sync_copy(dst4.at[wid, si], dst2d_v)

        def chunk_body(cj, carry2):
            @pl.when(si * SB + cj > 0)
            def _():
                for g in range(K // 16):
                    pltpu.make_async_copy(rows_v.at[pl.ds(g * 16, 16)],
                                          g_s.at[zeros16], ssem).wait()
            gdesc = pltpu.async_copy(wh_hbm.at[src2d_v.at[cj]],
                                     rows_v, gsem)
            exs = []
            dsts = []
            for g in range(K // 16):
                src16 = src2d_v[cj, pl.ds(g * 16, 16)]
                dst16 = dst2d_v[cj, pl.ds(g * 16, 16)]
                v1 = plsc.load_gather(s1_v, [src16])
                v2 = plsc.load_gather(s2_v, [dst16])
                t = v1 + v2
                e = jnp.where(t >= 0, t, t * 0.01)
                ex = jnp.exp(e - cvec)
                for j in range(16):
                    plsc.addupdate_scatter(den_v, [dst16], ex,
                                           mask=lanes == j)
                exs.append(ex)
                dsts.append(dst16)
            gdesc.wait()
            for g in range(K // 16):
                ex = exs[g]
                bjs = [lax.gather(
                    ex, jnp.full((16, 1), j, jnp.int32),
                    lax.GatherDimensionNumbers(
                        offset_dims=(), collapsed_slice_dims=(0,),
                        start_index_map=(0,)),
                    (1,),
                    mode=lax.GatherScatterMode.PROMISE_IN_BOUNDS)
                    for j in range(16)]
                for j in range(16):
                    r = g * 16 + j
                    for q in range(D // 16):
                        rows_v[r, pl.ds(q * 16, 16)] = (
                            rows_v[r, pl.ds(q * 16, 16)] * bjs[j])
                pltpu.async_copy(rows_v.at[pl.ds(g * 16, 16)],
                                 g_s.at[dsts[g]], ssem, add=True)
            return carry2

        lax.fori_loop(0, SB, chunk_body, 0)
        return carry

    lax.fori_loop(0, NSUPER, super_body, 0)
    for g in range(K // 16):
        pltpu.make_async_copy(rows_v.at[pl.ds(g * 16, 16)],
                              g_s.at[zeros16], ssem).wait()
    plsc.subcore_barrier()

    row0 = sid * RPT
    pltpu.sync_copy(g_s.at[pl.ds(row0, RPT)],
                    g_out.at[cid, pl.ds(row0, RPT)])
    pltpu.sync_copy(den_v, den_out.at[cid, sid])


_sc_mesh = plsc.VectorSubcoreMesh(core_axis_name="c", subcore_axis_name="s",
                                  num_cores=NC, num_subcores=NS)

_sc_kernel = functools.partial(
    pl.kernel,
    out_type=(jax.ShapeDtypeStruct((NC, NP, D), jnp.float32),
              jax.ShapeDtypeStruct((NC, NS, N), jnp.float32)),
    mesh=_sc_mesh,
    scratch_types=[
        pltpu.VMEM((SB, K), jnp.int32),
        pltpu.VMEM((SB, K), jnp.int32),
        pltpu.VMEM((N,), jnp.float32),
        pltpu.VMEM((N,), jnp.float32),
        pltpu.VMEM((128,), jnp.float32),
        pltpu.VMEM((K, D), jnp.float32),
        pltpu.VMEM((N,), jnp.float32),
        pltpu.SemaphoreType.DMA,
        pltpu.SemaphoreType.DMA,
        pltpu.VMEM_SHARED((NP, D), jnp.float32),
    ],
    compiler_params=pltpu.CompilerParams(needs_layout_passes=False),
)(_sc_body)


@jax.jit
def kernel(x, edge_index, W, b, a_w):
    src4 = edge_index[0].reshape(NW, NSUPER, SB, K)
    dst4 = edge_index[1].reshape(NW, NSUPER, SB, K)
    a2 = a_w.reshape(2, D)
    b2 = b.reshape(1, OUT)

    wh, s, c = pl.pallas_call(
        _t1_body,
        out_shape=[
            jax.ShapeDtypeStruct((N, OUT), jnp.float32),
            jax.ShapeDtypeStruct((2, N), jnp.float32),
            jax.ShapeDtypeStruct((1, 128), jnp.float32),
        ],
    )(x, W, b2, a2)

    s1 = s[0]
    s2 = s[1]
    c128 = c.reshape(128)
    zeros = jnp.zeros((RPT, D), dtype=jnp.float32)

    g, den = _sc_kernel(src4, dst4, s1, s2, c128, wh, zeros)

    BR = 1000
    h = pl.pallas_call(
        _t3_body,
        grid=(N // BR,),
        in_specs=[
            pl.BlockSpec((NC, BR, D), lambda i: (0, i, 0)),
            pl.BlockSpec((1, NC * NS, BR), lambda i: (i, 0, 0)),
        ],
        out_specs=pl.BlockSpec((BR, OUT), lambda i: (i, 0)),
        out_shape=jax.ShapeDtypeStruct((N, OUT), jnp.float32),
    )(g, den.reshape(NC * NS, N // BR, BR).transpose(1, 0, 2))
    return h
